# Initial kernel scaffold; baseline (speedup 1.0000x reference)
import functools

import jax
import jax.numpy as jnp
from jax import lax
from jax.experimental import pallas as pl
from jax.experimental.pallas import tpu as pltpu
from jax.experimental.pallas import tpu_sc as plsc

NC = 2
NS = 16
NW = NC * NS
LANES = 16
CH = 80

NUM_GRAPHS = 64


def _ceil_to(v, m):
  return -(-v // m) * m


def _sc_degree(dst3, n_pad, rpt):
  nch = dst3.shape[1]
  zch = rpt // 8

  mesh = plsc.VectorSubcoreMesh(
      core_axis_name="c", subcore_axis_name="s", num_cores=NC,
      num_subcores=NS)

  @functools.partial(
      pl.kernel,
      out_type=jax.ShapeDtypeStruct((NC, n_pad, 16), jnp.float32),
      mesh=mesh,
      scratch_types=[
          pltpu.VMEM((nch, CH), jnp.int32),
          pltpu.VMEM((CH, 16), jnp.float32),
          pltpu.VMEM((zch, 16), jnp.float32),
          pltpu.VMEM_SHARED((n_pad, 16), jnp.float32),
      ],
  )
  def deg_kernel(dst_hbm, out_hbm, idx_v, ones_v, z_v, acc):
    cid = lax.axis_index("c")
    sid = lax.axis_index("s")
    wid = sid * NC + cid

    ones16 = jnp.full((LANES,), 1.0, jnp.float32)
    zero16 = jnp.zeros((LANES,), jnp.float32)

    def fill_ones(r, _):
      ones_v[r, :] = ones16
      return 0

    lax.fori_loop(0, CH, fill_ones, 0)

    def fill_zero(r, _):
      z_v[r, :] = zero16
      return 0

    lax.fori_loop(0, zch, fill_zero, 0)

    def zero_acc(k, _):
      pltpu.sync_copy(z_v, acc.at[pl.ds(sid * rpt + k * zch, zch)])
      return 0

    lax.fori_loop(0, 8, zero_acc, 0)

    pltpu.sync_copy(dst_hbm.at[wid], idx_v)
    plsc.subcore_barrier()

    def body(j, _):
      pltpu.sync_copy(ones_v, acc.at[idx_v.at[j]], add=True)
      return 0

    lax.fori_loop(0, nch, body, 0)
    plsc.subcore_barrier()

    pltpu.sync_copy(acc.at[pl.ds(sid * rpt, rpt)],
                    out_hbm.at[cid, pl.ds(sid * rpt, rpt)])

  return deg_kernel(dst3)


def _sc_edge_aggregate(g, src3, dst3, n_pad, rpt):
  nch = src3.shape[1]
  d = g.shape[1]
  zch = rpt // 8

  mesh = plsc.VectorSubcoreMesh(
      core_axis_name="c", subcore_axis_name="s", num_cores=NC,
      num_subcores=NS)

  @functools.partial(
      pl.kernel,
      out_type=jax.ShapeDtypeStruct((NC, n_pad, d), jnp.float32),
      mesh=mesh,
      scratch_types=[
          pltpu.VMEM((nch, CH), jnp.int32),
          pltpu.VMEM((nch, CH), jnp.int32),
          pltpu.VMEM((CH, d), jnp.float32),
          pltpu.VMEM((CH, d), jnp.float32),
          pltpu.VMEM((zch, d), jnp.float32),
          pltpu.VMEM_SHARED((n_pad, d), jnp.float32),
          pltpu.SemaphoreType.DMA,
          pltpu.SemaphoreType.DMA,
      ],
  )
  def edge_kernel(g_hbm, src_hbm, dst_hbm, out_hbm, src_v, dst_v,
                  gbuf0, gbuf1, z_v, acc, sem0, sem1):
    cid = lax.axis_index("c")
    sid = lax.axis_index("s")
    wid = sid * NC + cid

    zero16 = jnp.zeros((LANES,), jnp.float32)

    def fill_zero(r, _):
      def inner(q, _):
        z_v[r, pl.ds(q * LANES, LANES)] = zero16
        return 0
      lax.fori_loop(0, d // LANES, inner, 0)
      return 0

    lax.fori_loop(0, zch, fill_zero, 0)

    def zero_acc(k, _):
      pltpu.sync_copy(z_v, acc.at[pl.ds(sid * rpt + k * zch, zch)])
      return 0

    lax.fori_loop(0, 8, zero_acc, 0)

    pltpu.sync_copy(src_hbm.at[wid], src_v)
    pltpu.sync_copy(dst_hbm.at[wid], dst_v)
    plsc.subcore_barrier()

    pltpu.async_copy(g_hbm.at[src_v.at[0]], gbuf0, sem0)

    def body(jj, _):
      j = jj * 2
      pltpu.make_async_copy(g_hbm.at[src_v.at[j]], gbuf0, sem0).wait()
      pltpu.async_copy(g_hbm.at[src_v.at[j + 1]], gbuf1, sem1)
      pltpu.sync_copy(gbuf0, acc.at[dst_v.at[j]], add=True)
      pltpu.make_async_copy(g_hbm.at[src_v.at[j + 1]], gbuf1, sem1).wait()

      @pl.when(jj + 1 < nch // 2)
      def _():
        pltpu.async_copy(g_hbm.at[src_v.at[j + 2]], gbuf0, sem0)

      pltpu.sync_copy(gbuf1, acc.at[dst_v.at[j + 1]], add=True)
      return 0

    lax.fori_loop(0, nch // 2, body, 0)
    plsc.subcore_barrier()

    pltpu.sync_copy(acc.at[pl.ds(sid * rpt, rpt)],
                    out_hbm.at[cid, pl.ds(sid * rpt, rpt)])

  return edge_kernel(g, src3, dst3)


ROWS = 1000


def _dinv_block(dp):
  return lax.rsqrt(dp[0, :, 0:1] + dp[1, :, 0:1] + 1.0)


def _tc_scale_matmul(x, w, dp):
  n, din = x.shape
  dout = w.shape[1]

  def body(x_ref, w_ref, dp_ref, g_ref):
    dinv = _dinv_block(dp_ref[...])
    h = jnp.dot(x_ref[...], w_ref[...], preferred_element_type=jnp.float32)
    g_ref[...] = h * dinv

  return pl.pallas_call(
      body,
      grid=(n // ROWS,),
      in_specs=[
          pl.BlockSpec((ROWS, din), lambda i: (i, 0)),
          pl.BlockSpec((din, dout), lambda i: (0, 0)),
          pl.BlockSpec((2, ROWS, 16), lambda i: (0, i, 0)),
      ],
      out_specs=pl.BlockSpec((ROWS, dout), lambda i: (i, 0)),
      out_shape=jax.ShapeDtypeStruct((n, dout), jnp.float32),
  )(x, w, dp)


def _tc_combine_matmul(acc, g1, dp, w):
  n, d = g1.shape
  dout = w.shape[1]

  def body(acc_ref, g1_ref, dp_ref, w_ref, x1_ref, g2_ref):
    dinv = _dinv_block(dp_ref[...])
    a = acc_ref[...]
    x1 = jnp.maximum((a[0] + a[1] + g1_ref[...]) * dinv, 0.0)
    x1_ref[...] = x1
    g2_ref[...] = jnp.dot(
        x1, w_ref[...], preferred_element_type=jnp.float32) * dinv

  return pl.pallas_call(
      body,
      grid=(n // ROWS,),
      in_specs=[
          pl.BlockSpec((2, ROWS, d), lambda i: (0, i, 0)),
          pl.BlockSpec((ROWS, d), lambda i: (i, 0)),
          pl.BlockSpec((2, ROWS, 16), lambda i: (0, i, 0)),
          pl.BlockSpec((d, dout), lambda i: (0, 0)),
      ],
      out_specs=[
          pl.BlockSpec((ROWS, d), lambda i: (i, 0)),
          pl.BlockSpec((ROWS, dout), lambda i: (i, 0)),
      ],
      out_shape=[
          jax.ShapeDtypeStruct((n, d), jnp.float32),
          jax.ShapeDtypeStruct((n, dout), jnp.float32),
      ],
  )(acc, g1, dp, w)


def _tc_combine_pool(acc, g2, dp, x1, batch2d):
  n, d = g2.shape

  def body(acc_ref, g2_ref, dp_ref, x1_ref, b_ref, out_ref):
    i = pl.program_id(0)

    @pl.when(i == 0)
    def _():
      out_ref[...] = jnp.full((NUM_GRAPHS, 2 * d), -jnp.inf, jnp.float32)

    dinv = _dinv_block(dp_ref[...])
    a = acc_ref[...]
    x2 = jnp.maximum((a[0] + a[1] + g2_ref[...]) * dinv, 0.0)
    x1 = x1_ref[...]
    b = b_ref[...]
    gmin = jnp.min(b)
    gmax = jnp.max(b)
    neg = jnp.float32(-jnp.inf)
    for gid in range(NUM_GRAPHS):

      @pl.when(jnp.logical_and(gmin <= gid, gid <= gmax))
      def _(gid=gid):
        m = b == gid
        r1 = jnp.max(jnp.where(m, x1, neg), axis=0, keepdims=True)
        r2 = jnp.max(jnp.where(m, x2, neg), axis=0, keepdims=True)
        row = jnp.concatenate([r1, r2], axis=1)
        out_ref[gid:gid + 1, :] = jnp.maximum(out_ref[gid:gid + 1, :], row)

  return pl.pallas_call(
      body,
      grid=(n // ROWS,),
      in_specs=[
          pl.BlockSpec((2, ROWS, d), lambda i: (0, i, 0)),
          pl.BlockSpec((ROWS, d), lambda i: (i, 0)),
          pl.BlockSpec((2, ROWS, 16), lambda i: (0, i, 0)),
          pl.BlockSpec((ROWS, d), lambda i: (i, 0)),
          pl.BlockSpec((ROWS, 1), lambda i: (i, 0)),
      ],
      out_specs=pl.BlockSpec((NUM_GRAPHS, 2 * d), lambda i: (0, 0)),
      out_shape=jax.ShapeDtypeStruct((NUM_GRAPHS, 2 * d), jnp.float32),
  )(acc, g2, dp, x1, batch2d)


def kernel(x, edge_index, batch, W1, W2):
  n = x.shape[0]
  e = edge_index.shape[1]

  ei = edge_index.astype(jnp.int32)
  src = ei[0]
  dst = ei[1]

  pad = (-e) % (NW * CH)
  if pad:
    src = jnp.concatenate([src, jnp.zeros((pad,), jnp.int32)])
    dst = jnp.concatenate([dst, jnp.full((pad,), n, jnp.int32)])
  nch = (e + pad) // (NW * CH)
  src3 = src.reshape(NW, nch, CH)
  dst3 = dst.reshape(NW, nch, CH)

  rpt = _ceil_to(n + (8 if pad else 0), NS * 8) // NS
  n_pad = rpt * NS

  dp = _sc_degree(dst3, n_pad, rpt)

  g1 = _tc_scale_matmul(x, W1, dp)
  acc1 = _sc_edge_aggregate(g1, src3, dst3, n_pad, rpt)
  x1, g2 = _tc_combine_matmul(acc1, g1, dp, W2)
  acc2 = _sc_edge_aggregate(g2, src3, dst3, n_pad, rpt)

  batch2d = batch.astype(jnp.int32).reshape(n, 1)
  return _tc_combine_pool(acc2, g2, dp, x1, batch2d)


# speedup vs baseline: 16.4997x; 16.4997x over previous
import functools

import jax
import jax.numpy as jnp
from jax import lax
from jax.experimental import pallas as pl
from jax.experimental.pallas import tpu as pltpu
from jax.experimental.pallas import tpu_sc as plsc

NC = 2
NS = 16
NW = NC * NS
LANES = 16
CH = 80

NUM_GRAPHS = 64


def _ceil_to(v, m):
  return -(-v // m) * m


def _sc_mesh():
  return plsc.VectorSubcoreMesh(
      core_axis_name="c", subcore_axis_name="s", num_cores=NC,
      num_subcores=NS)


def _sc_degree(dst3, n_pad, rpt):
  nch = dst3.shape[1]
  zch = rpt // 8

  @functools.partial(
      pl.kernel,
      out_type=jax.ShapeDtypeStruct((NC, n_pad, 16), jnp.float32),
      mesh=_sc_mesh(),
      scratch_types=[
          pltpu.VMEM((nch, CH), jnp.int32),
          pltpu.VMEM((CH, 16), jnp.float32),
          pltpu.VMEM((zch, 16), jnp.float32),
          pltpu.VMEM_SHARED((n_pad, 16), jnp.float32),
      ],
  )
  def deg_kernel(dst_hbm, out_hbm, idx_v, ones_v, z_v, acc):
    cid = lax.axis_index("c")
    sid = lax.axis_index("s")
    wid = sid * NC + cid

    ones16 = jnp.full((LANES,), 1.0, jnp.float32)
    zero16 = jnp.zeros((LANES,), jnp.float32)

    def fill_ones(r, _):
      ones_v[r, :] = ones16
      return 0

    lax.fori_loop(0, CH, fill_ones, 0)

    def fill_zero(r, _):
      z_v[r, :] = zero16
      return 0

    lax.fori_loop(0, zch, fill_zero, 0)

    def zero_acc(k, _):
      pltpu.sync_copy(z_v, acc.at[pl.ds(sid * rpt + k * zch, zch)])
      return 0

    lax.fori_loop(0, 8, zero_acc, 0)

    pltpu.sync_copy(dst_hbm.at[wid], idx_v)
    plsc.subcore_barrier()

    def body(j, _):
      pltpu.sync_copy(ones_v, acc.at[idx_v.at[j]], add=True)
      return 0

    lax.fori_loop(0, nch, body, 0)
    plsc.subcore_barrier()

    pltpu.sync_copy(acc.at[pl.ds(sid * rpt, rpt)],
                    out_hbm.at[cid, pl.ds(sid * rpt, rpt)])

  return deg_kernel(dst3)


def _sc_edge_aggregate(g, src3, dst3, n_pad, rpt):
  nch = src3.shape[1]
  d = g.shape[1]
  zch = rpt // 8

  @functools.partial(
      pl.kernel,
      out_type=jax.ShapeDtypeStruct((NC, n_pad, d), jnp.float32),
      mesh=_sc_mesh(),
      compiler_params=pltpu.CompilerParams(use_tc_tiling_on_sc=False),
      scratch_types=[
          pltpu.VMEM((nch, CH), jnp.int32),
          pltpu.VMEM((nch, CH), jnp.int32),
          pltpu.VMEM((CH, d), jnp.float32),
          pltpu.VMEM((CH, d), jnp.float32),
          pltpu.VMEM((zch, d), jnp.float32),
          pltpu.VMEM_SHARED((n_pad, d), jnp.float32),
          pltpu.SemaphoreType.DMA,
          pltpu.SemaphoreType.DMA,
      ],
  )
  def edge_kernel(g_hbm, src_hbm, dst_hbm, out_hbm, src_v, dst_v,
                  gbuf0, gbuf1, z_v, acc, sem0, sem1):
    cid = lax.axis_index("c")
    sid = lax.axis_index("s")
    wid = sid * NC + cid

    zero16 = jnp.zeros((LANES,), jnp.float32)

    def fill_zero(r, _):
      def inner(q, _):
        z_v[r, pl.ds(q * LANES, LANES)] = zero16
        return 0
      lax.fori_loop(0, d // LANES, inner, 0)
      return 0

    lax.fori_loop(0, zch, fill_zero, 0)

    def zero_acc(k, _):
      pltpu.sync_copy(z_v, acc.at[pl.ds(sid * rpt + k * zch, zch)])
      return 0

    lax.fori_loop(0, 8, zero_acc, 0)

    pltpu.sync_copy(src_hbm.at[wid], src_v)
    pltpu.sync_copy(dst_hbm.at[wid], dst_v)
    plsc.subcore_barrier()

    pltpu.async_copy(g_hbm.at[src_v.at[0]], gbuf0, sem0)

    def body(jj, _):
      j = jj * 2
      pltpu.make_async_copy(g_hbm.at[src_v.at[j]], gbuf0, sem0).wait()
      pltpu.async_copy(g_hbm.at[src_v.at[j + 1]], gbuf1, sem1)
      pltpu.sync_copy(gbuf0, acc.at[dst_v.at[j]], add=True)
      pltpu.make_async_copy(g_hbm.at[src_v.at[j + 1]], gbuf1, sem1).wait()

      @pl.when(jj + 1 < nch // 2)
      def _():
        pltpu.async_copy(g_hbm.at[src_v.at[j + 2]], gbuf0, sem0)

      pltpu.sync_copy(gbuf1, acc.at[dst_v.at[j + 1]], add=True)
      return 0

    lax.fori_loop(0, nch // 2, body, 0)

    if nch % 2 == 1:
      pltpu.async_copy(g_hbm.at[src_v.at[nch - 1]], gbuf0, sem0).wait()
      pltpu.sync_copy(gbuf0, acc.at[dst_v.at[nch - 1]], add=True)

    plsc.subcore_barrier()

    pltpu.sync_copy(acc.at[pl.ds(sid * rpt, rpt)],
                    out_hbm.at[cid, pl.ds(sid * rpt, rpt)])

  return edge_kernel(g, src3, dst3)


ROWS = 1000


def _dinv_block(dp):
  return lax.rsqrt(dp[0, :, 0:1] + dp[1, :, 0:1] + 1.0)


def _tc_scale_matmul(x, w, dp):
  n, din = x.shape
  dout = w.shape[1]
  dh = dout // 2

  def body(x_ref, w_ref, dp_ref, glo_ref, ghi_ref):
    dinv = _dinv_block(dp_ref[...])
    h = jnp.dot(x_ref[...], w_ref[...], preferred_element_type=jnp.float32)
    g = h * dinv
    glo_ref[...] = g[:, :dh]
    ghi_ref[...] = g[:, dh:]

  return pl.pallas_call(
      body,
      grid=(n // ROWS,),
      in_specs=[
          pl.BlockSpec((ROWS, din), lambda i: (i, 0)),
          pl.BlockSpec((din, dout), lambda i: (0, 0)),
          pl.BlockSpec((2, ROWS, 16), lambda i: (0, i, 0)),
      ],
      out_specs=[
          pl.BlockSpec((ROWS, dh), lambda i: (i, 0)),
          pl.BlockSpec((ROWS, dh), lambda i: (i, 0)),
      ],
      out_shape=[
          jax.ShapeDtypeStruct((n, dh), jnp.float32),
          jax.ShapeDtypeStruct((n, dh), jnp.float32),
      ],
  )(x, w, dp)


def _combine(al_ref, ah_ref, gl_ref, gh_ref, dinv):
  al = al_ref[...]
  ah = ah_ref[...]
  lo = (al[0] + al[1] + gl_ref[...]) * dinv
  hi = (ah[0] + ah[1] + gh_ref[...]) * dinv
  return jnp.maximum(jnp.concatenate([lo, hi], axis=1), 0.0)


def _tc_combine_matmul(accl, acch, g1l, g1h, dp, w):
  n, dh = g1l.shape
  d = 2 * dh
  dout = w.shape[1]

  def body(al_ref, ah_ref, g1l_ref, g1h_ref, dp_ref, w_ref,
           x1_ref, g2l_ref, g2h_ref):
    dinv = _dinv_block(dp_ref[...])
    x1 = _combine(al_ref, ah_ref, g1l_ref, g1h_ref, dinv)
    x1_ref[...] = x1
    g2 = jnp.dot(x1, w_ref[...], preferred_element_type=jnp.float32) * dinv
    g2l_ref[...] = g2[:, :dh]
    g2h_ref[...] = g2[:, dh:]

  return pl.pallas_call(
      body,
      grid=(n // ROWS,),
      in_specs=[
          pl.BlockSpec((2, ROWS, dh), lambda i: (0, i, 0)),
          pl.BlockSpec((2, ROWS, dh), lambda i: (0, i, 0)),
          pl.BlockSpec((ROWS, dh), lambda i: (i, 0)),
          pl.BlockSpec((ROWS, dh), lambda i: (i, 0)),
          pl.BlockSpec((2, ROWS, 16), lambda i: (0, i, 0)),
          pl.BlockSpec((d, dout), lambda i: (0, 0)),
      ],
      out_specs=[
          pl.BlockSpec((ROWS, d), lambda i: (i, 0)),
          pl.BlockSpec((ROWS, dh), lambda i: (i, 0)),
          pl.BlockSpec((ROWS, dh), lambda i: (i, 0)),
      ],
      out_shape=[
          jax.ShapeDtypeStruct((n, d), jnp.float32),
          jax.ShapeDtypeStruct((n, dh), jnp.float32),
          jax.ShapeDtypeStruct((n, dh), jnp.float32),
      ],
  )(accl, acch, g1l, g1h, dp, w)


def _tc_combine_pool(accl, acch, g2l, g2h, dp, x1, batch2d):
  n, dh = g2l.shape
  d = 2 * dh

  def body(al_ref, ah_ref, g2l_ref, g2h_ref, dp_ref, x1_ref, b_ref, out_ref):
    i = pl.program_id(0)

    @pl.when(i == 0)
    def _():
      out_ref[...] = jnp.full((NUM_GRAPHS, 2 * d), -jnp.inf, jnp.float32)

    dinv = _dinv_block(dp_ref[...])
    x2 = _combine(al_ref, ah_ref, g2l_ref, g2h_ref, dinv)
    x1 = x1_ref[...]
    b = b_ref[...]
    gmin = jnp.min(b)
    gmax = jnp.max(b)
    neg = jnp.float32(-jnp.inf)

    def gbody(gid, _):
      m = b == gid
      r1 = jnp.max(jnp.where(m, x1, neg), axis=0, keepdims=True)
      r2 = jnp.max(jnp.where(m, x2, neg), axis=0, keepdims=True)
      row = jnp.concatenate([r1, r2], axis=1)
      cur = out_ref[pl.ds(gid, 1), :]
      out_ref[pl.ds(gid, 1), :] = jnp.maximum(cur, row)
      return 0

    lax.fori_loop(gmin, gmax + 1, gbody, 0)

  return pl.pallas_call(
      body,
      grid=(n // ROWS,),
      in_specs=[
          pl.BlockSpec((2, ROWS, dh), lambda i: (0, i, 0)),
          pl.BlockSpec((2, ROWS, dh), lambda i: (0, i, 0)),
          pl.BlockSpec((ROWS, dh), lambda i: (i, 0)),
          pl.BlockSpec((ROWS, dh), lambda i: (i, 0)),
          pl.BlockSpec((2, ROWS, 16), lambda i: (0, i, 0)),
          pl.BlockSpec((ROWS, d), lambda i: (i, 0)),
          pl.BlockSpec((ROWS, 1), lambda i: (i, 0)),
      ],
      out_specs=pl.BlockSpec((NUM_GRAPHS, 2 * d), lambda i: (0, 0)),
      out_shape=jax.ShapeDtypeStruct((NUM_GRAPHS, 2 * d), jnp.float32),
  )(accl, acch, g2l, g2h, dp, x1, batch2d)


def kernel(x, edge_index, batch, W1, W2):
  n = x.shape[0]
  e = edge_index.shape[1]

  ei = edge_index.astype(jnp.int32)
  src = ei[0]
  dst = ei[1]

  pad = (-e) % (NW * CH)
  if pad:
    src = jnp.concatenate([src, jnp.zeros((pad,), jnp.int32)])
    dst = jnp.concatenate([dst, jnp.full((pad,), n, jnp.int32)])
  nch = (e + pad) // (NW * CH)
  src3 = src.reshape(NW, nch, CH)
  dst3 = dst.reshape(NW, nch, CH)

  rpt = _ceil_to(n + (8 if pad else 0), NS * 8) // NS
  n_pad = rpt * NS

  dp = _sc_degree(dst3, n_pad, rpt)

  g1l, g1h = _tc_scale_matmul(x, W1, dp)
  acc1l = _sc_edge_aggregate(g1l, src3, dst3, n_pad, rpt)
  acc1h = _sc_edge_aggregate(g1h, src3, dst3, n_pad, rpt)
  x1, g2l, g2h = _tc_combine_matmul(acc1l, acc1h, g1l, g1h, dp, W2)
  acc2l = _sc_edge_aggregate(g2l, src3, dst3, n_pad, rpt)
  acc2h = _sc_edge_aggregate(g2h, src3, dst3, n_pad, rpt)

  batch2d = batch.astype(jnp.int32).reshape(n, 1)
  return _tc_combine_pool(acc2l, acc2h, g2l, g2h, dp, x1, batch2d)


# speedup vs baseline: 27.4408x; 1.6631x over previous
import functools

import jax
import jax.numpy as jnp
from jax import lax
from jax.experimental import pallas as pl
from jax.experimental.pallas import tpu as pltpu
from jax.experimental.pallas import tpu_sc as plsc

NC = 2
NS = 16
NW = NC * NS
LANES = 16
CH = 80

NUM_GRAPHS = 64


def _ceil_to(v, m):
  return -(-v // m) * m


def _sc_mesh():
  return plsc.VectorSubcoreMesh(
      core_axis_name="c", subcore_axis_name="s", num_cores=NC,
      num_subcores=NS)


def _sc_degree(dst3, n_pad, rpt):
  nch = dst3.shape[1]
  zch = rpt // 8

  @functools.partial(
      pl.kernel,
      out_type=jax.ShapeDtypeStruct((NC, n_pad, 16), jnp.float32),
      mesh=_sc_mesh(),
      scratch_types=[
          pltpu.VMEM((nch, CH), jnp.int32),
          pltpu.VMEM((CH, 16), jnp.float32),
          pltpu.VMEM((zch, 16), jnp.float32),
          pltpu.VMEM_SHARED((n_pad, 16), jnp.float32),
      ],
  )
  def deg_kernel(dst_hbm, out_hbm, idx_v, ones_v, z_v, acc):
    cid = lax.axis_index("c")
    sid = lax.axis_index("s")
    wid = sid * NC + cid

    ones16 = jnp.full((LANES,), 1.0, jnp.float32)
    zero16 = jnp.zeros((LANES,), jnp.float32)

    def fill_ones(r, _):
      ones_v[r, :] = ones16
      return 0

    lax.fori_loop(0, CH, fill_ones, 0)

    def fill_zero(r, _):
      z_v[r, :] = zero16
      return 0

    lax.fori_loop(0, zch, fill_zero, 0)

    def zero_acc(k, _):
      pltpu.sync_copy(z_v, acc.at[pl.ds(sid * rpt + k * zch, zch)])
      return 0

    lax.fori_loop(0, 8, zero_acc, 0)

    pltpu.sync_copy(dst_hbm.at[wid], idx_v)
    plsc.subcore_barrier()

    def body(j, _):
      pltpu.sync_copy(ones_v, acc.at[idx_v.at[j]], add=True)
      return 0

    lax.fori_loop(0, nch, body, 0)
    plsc.subcore_barrier()

    pltpu.sync_copy(acc.at[pl.ds(sid * rpt, rpt)],
                    out_hbm.at[cid, pl.ds(sid * rpt, rpt)])

  return deg_kernel(dst3)


def _sc_edge_aggregate(g, src3, dst3, n_pad, rpt):
  nch = src3.shape[1]
  d = g.shape[1]
  zch = rpt // 8

  @functools.partial(
      pl.kernel,
      out_type=jax.ShapeDtypeStruct((NC, n_pad, d), jnp.float32),
      mesh=_sc_mesh(),
      compiler_params=pltpu.CompilerParams(use_tc_tiling_on_sc=False),
      scratch_types=[
          pltpu.VMEM((nch, CH), jnp.int32),
          pltpu.VMEM((nch, CH), jnp.int32),
          pltpu.VMEM((CH, d), jnp.float32),
          pltpu.VMEM((CH, d), jnp.float32),
          pltpu.VMEM((CH, d), jnp.float32),
          pltpu.VMEM((CH, d), jnp.float32),
          pltpu.VMEM((zch, d), jnp.float32),
          pltpu.VMEM_SHARED((n_pad, d), jnp.float32),
          [pltpu.SemaphoreType.DMA] * 4,
          [pltpu.SemaphoreType.DMA] * 4,
      ],
  )
  def edge_kernel(g_hbm, src_hbm, dst_hbm, out_hbm, src_v, dst_v,
                  gbuf0, gbuf1, gbuf2, gbuf3, z_v, acc, gsem, ssem):
    cid = lax.axis_index("c")
    sid = lax.axis_index("s")
    wid = sid * NC + cid

    zero16 = jnp.zeros((LANES,), jnp.float32)

    def fill_zero(r, _):
      def inner(q, _):
        z_v[r, pl.ds(q * LANES, LANES)] = zero16
        return 0
      lax.fori_loop(0, d // LANES, inner, 0)
      return 0

    lax.fori_loop(0, zch, fill_zero, 0)

    def zero_acc(k, _):
      pltpu.sync_copy(z_v, acc.at[pl.ds(sid * rpt + k * zch, zch)])
      return 0

    lax.fori_loop(0, 8, zero_acc, 0)

    pltpu.sync_copy(src_hbm.at[wid], src_v)
    pltpu.sync_copy(dst_hbm.at[wid], dst_v)
    plsc.subcore_barrier()

    bufs = (gbuf0, gbuf1, gbuf2, gbuf3)
    assert nch >= 4

    for b in range(3):
      pltpu.async_copy(g_hbm.at[src_v.at[b]], bufs[b], gsem[b])

    def body(jj0, _):
      for b in range(4):
        j = jj0 * 4 + b

        @pl.when(j < nch)
        def _(j=j, b=b):
          pltpu.make_async_copy(g_hbm.at[src_v.at[j]], bufs[b], gsem[b]).wait()
          bb = (b + 3) % 4

          @pl.when(j + 3 < nch)
          def _(j=j, bb=bb):
            @pl.when(j >= 1)
            def _(j=j, bb=bb):
              pltpu.make_async_copy(
                  bufs[bb], acc.at[dst_v.at[j - 1]], ssem[bb]).wait()

            pltpu.async_copy(g_hbm.at[src_v.at[j + 3]], bufs[bb], gsem[bb])

          pltpu.async_copy(bufs[b], acc.at[dst_v.at[j]], ssem[b], add=True)
      return 0

    lax.fori_loop(0, -(-nch // 4), body, 0)

    for jt in range(max(0, nch - 4), nch):
      pltpu.make_async_copy(
          bufs[jt % 4], acc.at[dst_v.at[jt]], ssem[jt % 4]).wait()

    plsc.subcore_barrier()

    pltpu.sync_copy(acc.at[pl.ds(sid * rpt, rpt)],
                    out_hbm.at[cid, pl.ds(sid * rpt, rpt)])

  return edge_kernel(g, src3, dst3)


ROWS = 1000


def _dinv_block(dp):
  return lax.rsqrt(dp[0, :, 0:1] + dp[1, :, 0:1] + 1.0)


def _tc_scale_matmul(x, w, dp):
  n, din = x.shape
  dout = w.shape[1]
  dh = dout // 2

  def body(x_ref, w_ref, dp_ref, glo_ref, ghi_ref):
    dinv = _dinv_block(dp_ref[...])
    h = jnp.dot(x_ref[...], w_ref[...], preferred_element_type=jnp.float32)
    g = h * dinv
    glo_ref[...] = g[:, :dh]
    ghi_ref[...] = g[:, dh:]

  return pl.pallas_call(
      body,
      grid=(n // ROWS,),
      in_specs=[
          pl.BlockSpec((ROWS, din), lambda i: (i, 0)),
          pl.BlockSpec((din, dout), lambda i: (0, 0)),
          pl.BlockSpec((2, ROWS, 16), lambda i: (0, i, 0)),
      ],
      out_specs=[
          pl.BlockSpec((ROWS, dh), lambda i: (i, 0)),
          pl.BlockSpec((ROWS, dh), lambda i: (i, 0)),
      ],
      out_shape=[
          jax.ShapeDtypeStruct((n, dh), jnp.float32),
          jax.ShapeDtypeStruct((n, dh), jnp.float32),
      ],
  )(x, w, dp)


def _combine(al_ref, ah_ref, gl_ref, gh_ref, dinv):
  al = al_ref[...]
  ah = ah_ref[...]
  lo = (al[0] + al[1] + gl_ref[...]) * dinv
  hi = (ah[0] + ah[1] + gh_ref[...]) * dinv
  return jnp.maximum(jnp.concatenate([lo, hi], axis=1), 0.0)


def _tc_combine_matmul(accl, acch, g1l, g1h, dp, w):
  n, dh = g1l.shape
  d = 2 * dh
  dout = w.shape[1]

  def body(al_ref, ah_ref, g1l_ref, g1h_ref, dp_ref, w_ref,
           x1_ref, g2l_ref, g2h_ref):
    dinv = _dinv_block(dp_ref[...])
    x1 = _combine(al_ref, ah_ref, g1l_ref, g1h_ref, dinv)
    x1_ref[...] = x1
    g2 = jnp.dot(x1, w_ref[...], preferred_element_type=jnp.float32) * dinv
    g2l_ref[...] = g2[:, :dh]
    g2h_ref[...] = g2[:, dh:]

  return pl.pallas_call(
      body,
      grid=(n // ROWS,),
      in_specs=[
          pl.BlockSpec((2, ROWS, dh), lambda i: (0, i, 0)),
          pl.BlockSpec((2, ROWS, dh), lambda i: (0, i, 0)),
          pl.BlockSpec((ROWS, dh), lambda i: (i, 0)),
          pl.BlockSpec((ROWS, dh), lambda i: (i, 0)),
          pl.BlockSpec((2, ROWS, 16), lambda i: (0, i, 0)),
          pl.BlockSpec((d, dout), lambda i: (0, 0)),
      ],
      out_specs=[
          pl.BlockSpec((ROWS, d), lambda i: (i, 0)),
          pl.BlockSpec((ROWS, dh), lambda i: (i, 0)),
          pl.BlockSpec((ROWS, dh), lambda i: (i, 0)),
      ],
      out_shape=[
          jax.ShapeDtypeStruct((n, d), jnp.float32),
          jax.ShapeDtypeStruct((n, dh), jnp.float32),
          jax.ShapeDtypeStruct((n, dh), jnp.float32),
      ],
  )(accl, acch, g1l, g1h, dp, w)


def _tc_combine_pool(accl, acch, g2l, g2h, dp, x1, batch2d):
  n, dh = g2l.shape
  d = 2 * dh

  def body(al_ref, ah_ref, g2l_ref, g2h_ref, dp_ref, x1_ref, b_ref, out_ref):
    i = pl.program_id(0)

    @pl.when(i == 0)
    def _():
      out_ref[...] = jnp.full((NUM_GRAPHS, 2 * d), -jnp.inf, jnp.float32)

    dinv = _dinv_block(dp_ref[...])
    x2 = _combine(al_ref, ah_ref, g2l_ref, g2h_ref, dinv)
    x1 = x1_ref[...]
    b = b_ref[...]
    gmin = jnp.min(b)
    gmax = jnp.max(b)
    neg = jnp.float32(-jnp.inf)

    def gbody(gid, _):
      m = b == gid
      r1 = jnp.max(jnp.where(m, x1, neg), axis=0, keepdims=True)
      r2 = jnp.max(jnp.where(m, x2, neg), axis=0, keepdims=True)
      row = jnp.concatenate([r1, r2], axis=1)
      cur = out_ref[pl.ds(gid, 1), :]
      out_ref[pl.ds(gid, 1), :] = jnp.maximum(cur, row)
      return 0

    lax.fori_loop(gmin, gmax + 1, gbody, 0)

  return pl.pallas_call(
      body,
      grid=(n // ROWS,),
      in_specs=[
          pl.BlockSpec((2, ROWS, dh), lambda i: (0, i, 0)),
          pl.BlockSpec((2, ROWS, dh), lambda i: (0, i, 0)),
          pl.BlockSpec((ROWS, dh), lambda i: (i, 0)),
          pl.BlockSpec((ROWS, dh), lambda i: (i, 0)),
          pl.BlockSpec((2, ROWS, 16), lambda i: (0, i, 0)),
          pl.BlockSpec((ROWS, d), lambda i: (i, 0)),
          pl.BlockSpec((ROWS, 1), lambda i: (i, 0)),
      ],
      out_specs=pl.BlockSpec((NUM_GRAPHS, 2 * d), lambda i: (0, 0)),
      out_shape=jax.ShapeDtypeStruct((NUM_GRAPHS, 2 * d), jnp.float32),
  )(accl, acch, g2l, g2h, dp, x1, batch2d)


def kernel(x, edge_index, batch, W1, W2):
  n = x.shape[0]
  e = edge_index.shape[1]

  ei = edge_index.astype(jnp.int32)
  src = ei[0]
  dst = ei[1]

  pad = (-e) % (NW * CH)
  if pad:
    src = jnp.concatenate([src, jnp.zeros((pad,), jnp.int32)])
    dst = jnp.concatenate([dst, jnp.full((pad,), n, jnp.int32)])
  nch = (e + pad) // (NW * CH)
  src3 = src.reshape(NW, nch, CH)
  dst3 = dst.reshape(NW, nch, CH)

  rpt = _ceil_to(n + (8 if pad else 0), NS * 8) // NS
  n_pad = rpt * NS

  dp = _sc_degree(dst3, n_pad, rpt)

  g1l, g1h = _tc_scale_matmul(x, W1, dp)
  acc1l = _sc_edge_aggregate(g1l, src3, dst3, n_pad, rpt)
  acc1h = _sc_edge_aggregate(g1h, src3, dst3, n_pad, rpt)
  x1, g2l, g2h = _tc_combine_matmul(acc1l, acc1h, g1l, g1h, dp, W2)
  acc2l = _sc_edge_aggregate(g2l, src3, dst3, n_pad, rpt)
  acc2h = _sc_edge_aggregate(g2h, src3, dst3, n_pad, rpt)

  batch2d = batch.astype(jnp.int32).reshape(n, 1)
  return _tc_combine_pool(acc2l, acc2h, g2l, g2h, dp, x1, batch2d)


# speedup vs baseline: 29.8865x; 1.0891x over previous
import functools

import jax
import jax.numpy as jnp
from jax import lax
from jax.experimental import pallas as pl
from jax.experimental.pallas import tpu as pltpu
from jax.experimental.pallas import tpu_sc as plsc

NC = 2
NS = 16
NW = NC * NS
LANES = 16
CH = 80

NUM_GRAPHS = 64


def _ceil_to(v, m):
  return -(-v // m) * m


def _sc_mesh():
  return plsc.VectorSubcoreMesh(
      core_axis_name="c", subcore_axis_name="s", num_cores=NC,
      num_subcores=NS)


def _sc_degree(dst3, n_pad, rpt):
  nch = dst3.shape[1]
  zch = rpt // 8

  @functools.partial(
      pl.kernel,
      out_type=jax.ShapeDtypeStruct((NC, n_pad, 16), jnp.float32),
      mesh=_sc_mesh(),
      scratch_types=[
          pltpu.VMEM((nch, CH), jnp.int32),
          pltpu.VMEM((CH, 16), jnp.float32),
          pltpu.VMEM((zch, 16), jnp.float32),
          pltpu.VMEM_SHARED((n_pad, 16), jnp.float32),
      ],
  )
  def deg_kernel(dst_hbm, out_hbm, idx_v, ones_v, z_v, acc):
    cid = lax.axis_index("c")
    sid = lax.axis_index("s")
    wid = sid * NC + cid

    ones16 = jnp.full((LANES,), 1.0, jnp.float32)
    zero16 = jnp.zeros((LANES,), jnp.float32)

    def fill_ones(r, _):
      ones_v[r, :] = ones16
      return 0

    lax.fori_loop(0, CH, fill_ones, 0)

    def fill_zero(r, _):
      z_v[r, :] = zero16
      return 0

    lax.fori_loop(0, zch, fill_zero, 0)

    def zero_acc(k, _):
      pltpu.sync_copy(z_v, acc.at[pl.ds(sid * rpt + k * zch, zch)])
      return 0

    lax.fori_loop(0, 8, zero_acc, 0)

    pltpu.sync_copy(dst_hbm.at[wid], idx_v)
    plsc.subcore_barrier()

    def body(j, _):
      pltpu.sync_copy(ones_v, acc.at[idx_v.at[j]], add=True)
      return 0

    lax.fori_loop(0, nch, body, 0)
    plsc.subcore_barrier()

    pltpu.sync_copy(acc.at[pl.ds(sid * rpt, rpt)],
                    out_hbm.at[cid, pl.ds(sid * rpt, rpt)])

  return deg_kernel(dst3)


def _sc_edge_aggregate(glo, ghi, src2, dst2, n_pad, rpt):
  nch = src2.shape[1]
  d = glo.shape[1]
  zch = rpt // 8

  @functools.partial(
      pl.kernel,
      out_type=jax.ShapeDtypeStruct((NC, n_pad, d), jnp.float32),
      mesh=_sc_mesh(),
      compiler_params=pltpu.CompilerParams(use_tc_tiling_on_sc=False),
      scratch_types=[
          pltpu.VMEM((nch, CH), jnp.int32),
          pltpu.VMEM((nch, CH), jnp.int32),
          pltpu.VMEM((CH, d), jnp.float32),
          pltpu.VMEM((CH, d), jnp.float32),
          pltpu.VMEM((CH, d), jnp.float32),
          pltpu.VMEM((CH, d), jnp.float32),
          pltpu.VMEM((zch, d), jnp.float32),
          pltpu.VMEM_SHARED((n_pad, d), jnp.float32),
          [pltpu.SemaphoreType.DMA] * 4,
          [pltpu.SemaphoreType.DMA] * 4,
      ],
  )
  def edge_kernel(glo_hbm, ghi_hbm, src_hbm, dst_hbm, out_hbm, src_v, dst_v,
                  gbuf0, gbuf1, gbuf2, gbuf3, z_v, acc, gsem, ssem):
    cid = lax.axis_index("c")
    sid = lax.axis_index("s")

    zero16 = jnp.zeros((LANES,), jnp.float32)

    def fill_zero(r, _):
      def inner(q, _):
        z_v[r, pl.ds(q * LANES, LANES)] = zero16
        return 0
      lax.fori_loop(0, d // LANES, inner, 0)
      return 0

    lax.fori_loop(0, zch, fill_zero, 0)

    def zero_acc(k, _):
      pltpu.sync_copy(z_v, acc.at[pl.ds(sid * rpt + k * zch, zch)])
      return 0

    lax.fori_loop(0, 8, zero_acc, 0)

    pltpu.sync_copy(src_hbm.at[sid], src_v)
    pltpu.sync_copy(dst_hbm.at[sid], dst_v)
    plsc.subcore_barrier()

    bufs = (gbuf0, gbuf1, gbuf2, gbuf3)
    assert nch >= 4

    def run(gtab):
      for b in range(3):
        pltpu.async_copy(gtab.at[src_v.at[b]], bufs[b], gsem[b])

      def body(jj0, _):
        for b in range(4):
          j = jj0 * 4 + b

          @pl.when(j < nch)
          def _(j=j, b=b):
            pltpu.make_async_copy(gtab.at[src_v.at[j]], bufs[b],
                                  gsem[b]).wait()
            bb = (b + 3) % 4

            @pl.when(j + 3 < nch)
            def _(j=j, bb=bb):
              @pl.when(j >= 1)
              def _(j=j, bb=bb):
                pltpu.make_async_copy(
                    bufs[bb], acc.at[dst_v.at[j - 1]], ssem[bb]).wait()

              pltpu.async_copy(gtab.at[src_v.at[j + 3]], bufs[bb], gsem[bb])

            pltpu.async_copy(bufs[b], acc.at[dst_v.at[j]], ssem[b], add=True)
        return 0

      lax.fori_loop(0, -(-nch // 4), body, 0)

      for jt in range(max(0, nch - 4), nch):
        pltpu.make_async_copy(
            bufs[jt % 4], acc.at[dst_v.at[jt]], ssem[jt % 4]).wait()

    @pl.when(cid == 0)
    def _():
      run(glo_hbm)

    @pl.when(cid == 1)
    def _():
      run(ghi_hbm)

    plsc.subcore_barrier()

    pltpu.sync_copy(acc.at[pl.ds(sid * rpt, rpt)],
                    out_hbm.at[cid, pl.ds(sid * rpt, rpt)])

  return edge_kernel(glo, ghi, src2, dst2)


ROWS = 1000


def _dinv_block(dp):
  return lax.rsqrt(dp[0, :, 0:1] + dp[1, :, 0:1] + 1.0)


def _tc_scale_matmul(x, w, dp):
  n, din = x.shape
  dout = w.shape[1]
  dh = dout // 2

  def body(x_ref, w_ref, dp_ref, glo_ref, ghi_ref):
    dinv = _dinv_block(dp_ref[...])
    h = jnp.dot(x_ref[...], w_ref[...], preferred_element_type=jnp.float32)
    g = h * dinv
    glo_ref[...] = g[:, :dh]
    ghi_ref[...] = g[:, dh:]

  return pl.pallas_call(
      body,
      grid=(n // ROWS,),
      in_specs=[
          pl.BlockSpec((ROWS, din), lambda i: (i, 0)),
          pl.BlockSpec((din, dout), lambda i: (0, 0)),
          pl.BlockSpec((2, ROWS, 16), lambda i: (0, i, 0)),
      ],
      out_specs=[
          pl.BlockSpec((ROWS, dh), lambda i: (i, 0)),
          pl.BlockSpec((ROWS, dh), lambda i: (i, 0)),
      ],
      out_shape=[
          jax.ShapeDtypeStruct((n, dh), jnp.float32),
          jax.ShapeDtypeStruct((n, dh), jnp.float32),
      ],
  )(x, w, dp)


def _combine(a_ref, gl_ref, gh_ref, dinv):
  a = a_ref[...]
  lo = (a[0] + gl_ref[...]) * dinv
  hi = (a[1] + gh_ref[...]) * dinv
  return jnp.maximum(jnp.concatenate([lo, hi], axis=1), 0.0)


def _tc_combine_matmul(agg, g1l, g1h, dp, w):
  n, dh = g1l.shape
  d = 2 * dh
  dout = w.shape[1]

  def body(a_ref, g1l_ref, g1h_ref, dp_ref, w_ref, x1_ref, g2l_ref, g2h_ref):
    dinv = _dinv_block(dp_ref[...])
    x1 = _combine(a_ref, g1l_ref, g1h_ref, dinv)
    x1_ref[...] = x1
    g2 = jnp.dot(x1, w_ref[...], preferred_element_type=jnp.float32) * dinv
    g2l_ref[...] = g2[:, :dh]
    g2h_ref[...] = g2[:, dh:]

  return pl.pallas_call(
      body,
      grid=(n // ROWS,),
      in_specs=[
          pl.BlockSpec((2, ROWS, dh), lambda i: (0, i, 0)),
          pl.BlockSpec((ROWS, dh), lambda i: (i, 0)),
          pl.BlockSpec((ROWS, dh), lambda i: (i, 0)),
          pl.BlockSpec((2, ROWS, 16), lambda i: (0, i, 0)),
          pl.BlockSpec((d, dout), lambda i: (0, 0)),
      ],
      out_specs=[
          pl.BlockSpec((ROWS, d), lambda i: (i, 0)),
          pl.BlockSpec((ROWS, dh), lambda i: (i, 0)),
          pl.BlockSpec((ROWS, dh), lambda i: (i, 0)),
      ],
      out_shape=[
          jax.ShapeDtypeStruct((n, d), jnp.float32),
          jax.ShapeDtypeStruct((n, dh), jnp.float32),
          jax.ShapeDtypeStruct((n, dh), jnp.float32),
      ],
  )(agg, g1l, g1h, dp, w)


def _tc_combine_pool(agg, g2l, g2h, dp, x1, batch2d):
  n, dh = g2l.shape
  d = 2 * dh

  def body(a_ref, g2l_ref, g2h_ref, dp_ref, x1_ref, b_ref, out_ref):
    i = pl.program_id(0)

    @pl.when(i == 0)
    def _():
      out_ref[...] = jnp.full((NUM_GRAPHS, 2 * d), -jnp.inf, jnp.float32)

    dinv = _dinv_block(dp_ref[...])
    x2 = _combine(a_ref, g2l_ref, g2h_ref, dinv)
    x1 = x1_ref[...]
    b = b_ref[...]
    gmin = jnp.min(b)
    gmax = jnp.max(b)
    neg = jnp.float32(-jnp.inf)

    def gbody(gid, _):
      m = b == gid
      r1 = jnp.max(jnp.where(m, x1, neg), axis=0, keepdims=True)
      r2 = jnp.max(jnp.where(m, x2, neg), axis=0, keepdims=True)
      row = jnp.concatenate([r1, r2], axis=1)
      cur = out_ref[pl.ds(gid, 1), :]
      out_ref[pl.ds(gid, 1), :] = jnp.maximum(cur, row)
      return 0

    lax.fori_loop(gmin, gmax + 1, gbody, 0)

  return pl.pallas_call(
      body,
      grid=(n // ROWS,),
      in_specs=[
          pl.BlockSpec((2, ROWS, dh), lambda i: (0, i, 0)),
          pl.BlockSpec((ROWS, dh), lambda i: (i, 0)),
          pl.BlockSpec((ROWS, dh), lambda i: (i, 0)),
          pl.BlockSpec((2, ROWS, 16), lambda i: (0, i, 0)),
          pl.BlockSpec((ROWS, d), lambda i: (i, 0)),
          pl.BlockSpec((ROWS, 1), lambda i: (i, 0)),
      ],
      out_specs=pl.BlockSpec((NUM_GRAPHS, 2 * d), lambda i: (0, 0)),
      out_shape=jax.ShapeDtypeStruct((NUM_GRAPHS, 2 * d), jnp.float32),
  )(agg, g2l, g2h, dp, x1, batch2d)


def kernel(x, edge_index, batch, W1, W2):
  n = x.shape[0]
  e = edge_index.shape[1]

  ei = edge_index.astype(jnp.int32)
  src = ei[0]
  dst = ei[1]

  pad = (-e) % (NS * CH)
  if pad:
    src = jnp.concatenate([src, jnp.zeros((pad,), jnp.int32)])
    dst = jnp.concatenate([dst, jnp.full((pad,), n, jnp.int32)])
  ep = e + pad

  pad32 = (-e) % (NW * CH)
  src32 = src
  dst32 = dst
  if pad32 != pad:
    extra = pad32 - pad
    src32 = jnp.concatenate([src, jnp.zeros((extra,), jnp.int32)])
    dst32 = jnp.concatenate([dst, jnp.full((extra,), n, jnp.int32)])
  nch32 = (e + pad32) // (NW * CH)
  dst3 = dst32.reshape(NW, nch32, CH)
  del src32

  nchr = ep // (NS * CH)
  src2 = src.reshape(NS, nchr, CH)
  dst2 = dst.reshape(NS, nchr, CH)

  rpt = _ceil_to(n + (8 if (pad or pad32) else 0), NS * 8) // NS
  n_pad = rpt * NS

  dp = _sc_degree(dst3, n_pad, rpt)

  g1l, g1h = _tc_scale_matmul(x, W1, dp)
  agg1 = _sc_edge_aggregate(g1l, g1h, src2, dst2, n_pad, rpt)
  x1, g2l, g2h = _tc_combine_matmul(agg1, g1l, g1h, dp, W2)
  agg2 = _sc_edge_aggregate(g2l, g2h, src2, dst2, n_pad, rpt)

  batch2d = batch.astype(jnp.int32).reshape(n, 1)
  return _tc_combine_pool(agg2, g2l, g2h, dp, x1, batch2d)


# speedup vs baseline: 30.8429x; 1.0320x over previous
import functools

import jax
import jax.numpy as jnp
from jax import lax
from jax.experimental import pallas as pl
from jax.experimental.pallas import tpu as pltpu
from jax.experimental.pallas import tpu_sc as plsc

NC = 2
NS = 16
NW = NC * NS
LANES = 16
CH = 80

NUM_GRAPHS = 64


def _ceil_to(v, m):
  return -(-v // m) * m


def _sc_mesh():
  return plsc.VectorSubcoreMesh(
      core_axis_name="c", subcore_axis_name="s", num_cores=NC,
      num_subcores=NS)


def _sc_degree(dst3, n_pad, rpt):
  nch = dst3.shape[1]
  zch = rpt // 8

  @functools.partial(
      pl.kernel,
      out_type=jax.ShapeDtypeStruct((NC, n_pad, 16), jnp.float32),
      mesh=_sc_mesh(),
      scratch_types=[
          pltpu.VMEM((nch, CH), jnp.int32),
          pltpu.VMEM((CH, 16), jnp.float32),
          pltpu.VMEM((zch, 16), jnp.float32),
          pltpu.VMEM_SHARED((n_pad, 16), jnp.float32),
      ],
  )
  def deg_kernel(dst_hbm, out_hbm, idx_v, ones_v, z_v, acc):
    cid = lax.axis_index("c")
    sid = lax.axis_index("s")
    wid = sid * NC + cid

    ones16 = jnp.full((LANES,), 1.0, jnp.float32)
    zero16 = jnp.zeros((LANES,), jnp.float32)

    def fill_ones(r, _):
      ones_v[r, :] = ones16
      return 0

    lax.fori_loop(0, CH, fill_ones, 0)

    def fill_zero(r, _):
      z_v[r, :] = zero16
      return 0

    lax.fori_loop(0, zch, fill_zero, 0)

    def zero_acc(k, _):
      pltpu.sync_copy(z_v, acc.at[pl.ds(sid * rpt + k * zch, zch)])
      return 0

    lax.fori_loop(0, 8, zero_acc, 0)

    pltpu.sync_copy(dst_hbm.at[wid], idx_v)
    plsc.subcore_barrier()

    def body(j, _):
      pltpu.sync_copy(ones_v, acc.at[idx_v.at[j]], add=True)
      return 0

    lax.fori_loop(0, nch, body, 0)
    plsc.subcore_barrier()

    pltpu.sync_copy(acc.at[pl.ds(sid * rpt, rpt)],
                    out_hbm.at[cid, pl.ds(sid * rpt, rpt)])

  return deg_kernel(dst3)


def _sc_edge_aggregate(glo, ghi, src2, dst2, n_pad, rpt):
  nch = src2.shape[1]
  d = glo.shape[1]
  zch = rpt // 8

  @functools.partial(
      pl.kernel,
      out_type=jax.ShapeDtypeStruct((NC, n_pad, d), jnp.float32),
      mesh=_sc_mesh(),
      compiler_params=pltpu.CompilerParams(use_tc_tiling_on_sc=False),
      scratch_types=[
          pltpu.VMEM((nch, CH), jnp.int32),
          pltpu.VMEM((nch, CH), jnp.int32),
          pltpu.VMEM((CH, d), jnp.float32),
          pltpu.VMEM((CH, d), jnp.float32),
          pltpu.VMEM((CH, d), jnp.float32),
          pltpu.VMEM((CH, d), jnp.float32),
          pltpu.VMEM((CH, d), jnp.float32),
          pltpu.VMEM((CH, d), jnp.float32),
          pltpu.VMEM((zch, d), jnp.float32),
          pltpu.VMEM_SHARED((n_pad, d), jnp.float32),
          [pltpu.SemaphoreType.DMA] * 6,
          [pltpu.SemaphoreType.DMA] * 6,
      ],
  )
  def edge_kernel(glo_hbm, ghi_hbm, src_hbm, dst_hbm, out_hbm, src_v, dst_v,
                  gbuf0, gbuf1, gbuf2, gbuf3, gbuf4, gbuf5, z_v, acc,
                  gsem, ssem):
    cid = lax.axis_index("c")
    sid = lax.axis_index("s")

    zero16 = jnp.zeros((LANES,), jnp.float32)

    def fill_zero(r, _):
      def inner(q, _):
        z_v[r, pl.ds(q * LANES, LANES)] = zero16
        return 0
      lax.fori_loop(0, d // LANES, inner, 0)
      return 0

    lax.fori_loop(0, zch, fill_zero, 0)

    def zero_acc(k, _):
      pltpu.sync_copy(z_v, acc.at[pl.ds(sid * rpt + k * zch, zch)])
      return 0

    lax.fori_loop(0, 8, zero_acc, 0)

    pltpu.sync_copy(src_hbm.at[sid], src_v)
    pltpu.sync_copy(dst_hbm.at[sid], dst_v)
    plsc.subcore_barrier()

    NB = 6
    PF = NB - 1
    bufs = (gbuf0, gbuf1, gbuf2, gbuf3, gbuf4, gbuf5)
    assert nch >= NB

    def run(gtab):
      for b in range(PF):
        pltpu.async_copy(gtab.at[src_v.at[b]], bufs[b], gsem[b])

      def body(jj0, _):
        for b in range(NB):
          j = jj0 * NB + b

          @pl.when(j < nch)
          def _(j=j, b=b):
            pltpu.make_async_copy(gtab.at[src_v.at[j]], bufs[b],
                                  gsem[b]).wait()
            bb = (b + PF) % NB

            @pl.when(j + PF < nch)
            def _(j=j, bb=bb):
              @pl.when(j >= 1)
              def _(j=j, bb=bb):
                pltpu.make_async_copy(
                    bufs[bb], acc.at[dst_v.at[j - 1]], ssem[bb]).wait()

              pltpu.async_copy(gtab.at[src_v.at[j + PF]], bufs[bb], gsem[bb])

            pltpu.async_copy(bufs[b], acc.at[dst_v.at[j]], ssem[b], add=True)
        return 0

      lax.fori_loop(0, -(-nch // NB), body, 0)

      for jt in range(max(0, nch - NB), nch):
        pltpu.make_async_copy(
            bufs[jt % NB], acc.at[dst_v.at[jt]], ssem[jt % NB]).wait()

    @pl.when(cid == 0)
    def _():
      run(glo_hbm)

    @pl.when(cid == 1)
    def _():
      run(ghi_hbm)

    plsc.subcore_barrier()

    pltpu.sync_copy(acc.at[pl.ds(sid * rpt, rpt)],
                    out_hbm.at[cid, pl.ds(sid * rpt, rpt)])

  return edge_kernel(glo, ghi, src2, dst2)


ROWS = 1000


def _dinv_block(dp):
  return lax.rsqrt(dp[0, :, 0:1] + dp[1, :, 0:1] + 1.0)


def _tc_matmul(x, w):
  n, din = x.shape
  dout = w.shape[1]

  def body(x_ref, w_ref, h_ref):
    h_ref[...] = jnp.dot(x_ref[...], w_ref[...],
                         preferred_element_type=jnp.float32)

  return pl.pallas_call(
      body,
      grid=(n // ROWS,),
      in_specs=[
          pl.BlockSpec((ROWS, din), lambda i: (i, 0)),
          pl.BlockSpec((din, dout), lambda i: (0, 0)),
      ],
      out_specs=pl.BlockSpec((ROWS, dout), lambda i: (i, 0)),
      out_shape=jax.ShapeDtypeStruct((n, dout), jnp.float32),
  )(x, w)


def _tc_scale(h, dp):
  n, dout = h.shape
  dh = dout // 2

  def body(h_ref, dp_ref, glo_ref, ghi_ref):
    dinv = _dinv_block(dp_ref[...])
    g = h_ref[...] * dinv
    glo_ref[...] = g[:, :dh]
    ghi_ref[...] = g[:, dh:]

  return pl.pallas_call(
      body,
      grid=(n // ROWS,),
      in_specs=[
          pl.BlockSpec((ROWS, dout), lambda i: (i, 0)),
          pl.BlockSpec((2, ROWS, 16), lambda i: (0, i, 0)),
      ],
      out_specs=[
          pl.BlockSpec((ROWS, dh), lambda i: (i, 0)),
          pl.BlockSpec((ROWS, dh), lambda i: (i, 0)),
      ],
      out_shape=[
          jax.ShapeDtypeStruct((n, dh), jnp.float32),
          jax.ShapeDtypeStruct((n, dh), jnp.float32),
      ],
  )(h, dp)


def _combine(a_ref, gl_ref, gh_ref, dinv):
  a = a_ref[...]
  lo = (a[0] + gl_ref[...]) * dinv
  hi = (a[1] + gh_ref[...]) * dinv
  return jnp.maximum(jnp.concatenate([lo, hi], axis=1), 0.0)


def _tc_combine_matmul(agg, g1l, g1h, dp, w):
  n, dh = g1l.shape
  d = 2 * dh
  dout = w.shape[1]

  def body(a_ref, g1l_ref, g1h_ref, dp_ref, w_ref, x1_ref, g2l_ref, g2h_ref):
    dinv = _dinv_block(dp_ref[...])
    x1 = _combine(a_ref, g1l_ref, g1h_ref, dinv)
    x1_ref[...] = x1
    g2 = jnp.dot(x1, w_ref[...], preferred_element_type=jnp.float32) * dinv
    g2l_ref[...] = g2[:, :dh]
    g2h_ref[...] = g2[:, dh:]

  return pl.pallas_call(
      body,
      grid=(n // ROWS,),
      in_specs=[
          pl.BlockSpec((2, ROWS, dh), lambda i: (0, i, 0)),
          pl.BlockSpec((ROWS, dh), lambda i: (i, 0)),
          pl.BlockSpec((ROWS, dh), lambda i: (i, 0)),
          pl.BlockSpec((2, ROWS, 16), lambda i: (0, i, 0)),
          pl.BlockSpec((d, dout), lambda i: (0, 0)),
      ],
      out_specs=[
          pl.BlockSpec((ROWS, d), lambda i: (i, 0)),
          pl.BlockSpec((ROWS, dh), lambda i: (i, 0)),
          pl.BlockSpec((ROWS, dh), lambda i: (i, 0)),
      ],
      out_shape=[
          jax.ShapeDtypeStruct((n, d), jnp.float32),
          jax.ShapeDtypeStruct((n, dh), jnp.float32),
          jax.ShapeDtypeStruct((n, dh), jnp.float32),
      ],
  )(agg, g1l, g1h, dp, w)


def _tc_combine_pool(agg, g2l, g2h, dp, x1, batch2d):
  n, dh = g2l.shape
  d = 2 * dh

  def body(a_ref, g2l_ref, g2h_ref, dp_ref, x1_ref, b_ref, out_ref):
    i = pl.program_id(0)

    @pl.when(i == 0)
    def _():
      out_ref[...] = jnp.full((NUM_GRAPHS, 2 * d), -jnp.inf, jnp.float32)

    dinv = _dinv_block(dp_ref[...])
    x2 = _combine(a_ref, g2l_ref, g2h_ref, dinv)
    x1 = x1_ref[...]
    b = b_ref[...]
    gmin = jnp.min(b)
    gmax = jnp.max(b)
    neg = jnp.float32(-jnp.inf)

    def gbody(gid, _):
      m = b == gid
      r1 = jnp.max(jnp.where(m, x1, neg), axis=0, keepdims=True)
      r2 = jnp.max(jnp.where(m, x2, neg), axis=0, keepdims=True)
      row = jnp.concatenate([r1, r2], axis=1)
      cur = out_ref[pl.ds(gid, 1), :]
      out_ref[pl.ds(gid, 1), :] = jnp.maximum(cur, row)
      return 0

    lax.fori_loop(gmin, gmax + 1, gbody, 0)

  return pl.pallas_call(
      body,
      grid=(n // ROWS,),
      in_specs=[
          pl.BlockSpec((2, ROWS, dh), lambda i: (0, i, 0)),
          pl.BlockSpec((ROWS, dh), lambda i: (i, 0)),
          pl.BlockSpec((ROWS, dh), lambda i: (i, 0)),
          pl.BlockSpec((2, ROWS, 16), lambda i: (0, i, 0)),
          pl.BlockSpec((ROWS, d), lambda i: (i, 0)),
          pl.BlockSpec((ROWS, 1), lambda i: (i, 0)),
      ],
      out_specs=pl.BlockSpec((NUM_GRAPHS, 2 * d), lambda i: (0, 0)),
      out_shape=jax.ShapeDtypeStruct((NUM_GRAPHS, 2 * d), jnp.float32),
  )(agg, g2l, g2h, dp, x1, batch2d)


def kernel(x, edge_index, batch, W1, W2):
  n = x.shape[0]
  e = edge_index.shape[1]

  ei = edge_index.astype(jnp.int32)
  src = ei[0]
  dst = ei[1]

  pad = (-e) % (NS * CH)
  if pad:
    src = jnp.concatenate([src, jnp.zeros((pad,), jnp.int32)])
    dst = jnp.concatenate([dst, jnp.full((pad,), n, jnp.int32)])
  ep = e + pad

  pad32 = (-e) % (NW * CH)
  src32 = src
  dst32 = dst
  if pad32 != pad:
    extra = pad32 - pad
    src32 = jnp.concatenate([src, jnp.zeros((extra,), jnp.int32)])
    dst32 = jnp.concatenate([dst, jnp.full((extra,), n, jnp.int32)])
  nch32 = (e + pad32) // (NW * CH)
  dst3 = dst32.reshape(NW, nch32, CH)
  del src32

  nchr = ep // (NS * CH)
  src2 = src.reshape(NS, nchr, CH)
  dst2 = dst.reshape(NS, nchr, CH)

  rpt = _ceil_to(n + (8 if (pad or pad32) else 0), NS * 8) // NS
  n_pad = rpt * NS

  h1 = _tc_matmul(x, W1)
  dp = _sc_degree(dst3, n_pad, rpt)
  g1l, g1h = _tc_scale(h1, dp)
  agg1 = _sc_edge_aggregate(g1l, g1h, src2, dst2, n_pad, rpt)
  x1, g2l, g2h = _tc_combine_matmul(agg1, g1l, g1h, dp, W2)
  agg2 = _sc_edge_aggregate(g2l, g2h, src2, dst2, n_pad, rpt)

  batch2d = batch.astype(jnp.int32).reshape(n, 1)
  return _tc_combine_pool(agg2, g2l, g2h, dp, x1, batch2d)
